# baseline (speedup 1.0000x reference)
import jax
import jax.numpy as jnp
from jax import lax
from jax.experimental import pallas as pl

_N = 4096
_K = 30
_NEG = -1e15
_R = 256
_CHUNK = 32
_NCHUNK = _N // _CHUNK


def _monotone(v):
    b = lax.bitcast_convert_type(v, jnp.int32)
    return b ^ jnp.bitwise_and(jnp.right_shift(b, 31), jnp.int32(0x7FFFFFFF))


def _topk_softmax_kernel(x_ref, o_ref):
    min32 = jnp.int32(-(2**31))
    x = x_ref[...]
    key = _monotone(x)

    def make_search(data, k):
        def run(nsteps, w0):
            def body(i, w):
                bit = jnp.left_shift(jnp.int32(1), jnp.int32(nsteps) - 1 - i)
                cand_w = jnp.bitwise_or(w, bit)
                cand_t = jnp.bitwise_xor(cand_w, min32)
                cnt = jnp.sum((data >= cand_t).astype(jnp.float32), axis=1,
                              keepdims=True)
                return jnp.where(cnt >= k, cand_w, w)

            return lax.fori_loop(0, nsteps, body, w0, unroll=8)

        return run

    cm = jnp.max(x.reshape(_R, _NCHUNK, _CHUNK), axis=2)
    keyc = _monotone(cm)
    w_t = make_search(keyc, _K)(32, jnp.zeros((_R, 1), jnp.int32))
    u_m = jnp.bitwise_xor(jnp.max(keyc, axis=1, keepdims=True), min32)
    d = jnp.bitwise_xor(w_t, u_m)
    df = lax.bitcast_convert_type(d.astype(jnp.float32), jnp.int32)
    nbits = jnp.right_shift(df, 23) - 126
    nbits = jnp.where(d < 0, jnp.int32(32), jnp.clip(nbits, 0, 32))
    shift = jnp.minimum(nbits, 31)
    pmask = jnp.where(
        nbits >= 32, jnp.int32(0),
        ~(jnp.left_shift(jnp.int32(1), shift) - 1))
    w0 = jnp.bitwise_and(w_t, pmask)
    maxnb = jnp.max(nbits)

    search = make_search(key, _K)
    w = lax.cond(maxnb <= 24,
                 lambda a: search(24, a),
                 lambda a: search(32, a),
                 w0)
    kth = jnp.bitwise_xor(w, min32)

    keep = (key >= kth) & (x > 0.0)
    m = jnp.where(keep, x, _NEG)
    rowmax = jnp.max(m, axis=1, keepdims=True)
    e = jnp.exp(m - rowmax)
    s = jnp.sum(e, axis=1, keepdims=True)
    o_ref[...] = e / s


def kernel(new_supports):
    n = new_supports.shape[0]
    return pl.pallas_call(
        _topk_softmax_kernel,
        grid=(n // _R,),
        in_specs=[pl.BlockSpec((_R, _N), lambda i: (i, 0))],
        out_specs=pl.BlockSpec((_R, _N), lambda i: (i, 0)),
        out_shape=jax.ShapeDtypeStruct((n, _N), jnp.float32),
    )(new_supports)


# speedup vs baseline: 1.0283x; 1.0283x over previous
import jax
import jax.numpy as jnp
from jax import lax
from jax.experimental import pallas as pl

_N = 4096
_K = 30
_NEG = -1e15
_R = 256
_CHUNK = 32
_NCHUNK = _N // _CHUNK


def _monotone(v):
    b = lax.bitcast_convert_type(v, jnp.int32)
    return b ^ jnp.bitwise_and(jnp.right_shift(b, 31), jnp.int32(0x7FFFFFFF))


def _topk_softmax_kernel(x_ref, o_ref):
    min32 = jnp.int32(-(2**31))
    x = x_ref[...]
    key = _monotone(x)

    def make_search(data, k):
        def run(nsteps, w0):
            def body(i, w):
                bit = jnp.left_shift(jnp.int32(1), jnp.int32(nsteps) - 1 - i)
                cand_w = jnp.bitwise_or(w, bit)
                cand_t = jnp.bitwise_xor(cand_w, min32)
                cnt = jnp.sum((data >= cand_t).astype(jnp.float32), axis=1,
                              keepdims=True)
                return jnp.where(cnt >= k, cand_w, w)

            return lax.fori_loop(0, nsteps, body, w0, unroll=8)

        return run

    cm = jnp.max(x.reshape(_R, _NCHUNK, _CHUNK), axis=2)
    keyc = _monotone(cm)
    w_t = make_search(keyc, _K)(32, jnp.zeros((_R, 1), jnp.int32))
    u_m = jnp.bitwise_xor(jnp.max(keyc, axis=1, keepdims=True), min32)
    d = jnp.bitwise_xor(w_t, u_m)
    df = lax.bitcast_convert_type(d.astype(jnp.float32), jnp.int32)
    nbits = jnp.right_shift(df, 23) - 126
    nbits = jnp.where(d < 0, jnp.int32(32), jnp.clip(nbits, 0, 32))
    shift = jnp.minimum(nbits, 31)
    pmask = jnp.where(
        nbits >= 32, jnp.int32(0),
        ~(jnp.left_shift(jnp.int32(1), shift) - 1))
    w0 = jnp.bitwise_and(w_t, pmask)
    maxnb = jnp.max(nbits)

    del maxnb
    search = make_search(key, _K)
    w = search(32, w0)
    kth = jnp.bitwise_xor(w, min32)

    keep = (key >= kth) & (x > 0.0)
    m = jnp.where(keep, x, _NEG)
    rowmax = jnp.max(m, axis=1, keepdims=True)
    e = jnp.exp(m - rowmax)
    s = jnp.sum(e, axis=1, keepdims=True)
    o_ref[...] = e / s


def kernel(new_supports):
    n = new_supports.shape[0]
    return pl.pallas_call(
        _topk_softmax_kernel,
        grid=(n // _R,),
        in_specs=[pl.BlockSpec((_R, _N), lambda i: (i, 0))],
        out_specs=pl.BlockSpec((_R, _N), lambda i: (i, 0)),
        out_shape=jax.ShapeDtypeStruct((n, _N), jnp.float32),
    )(new_supports)


# speedup vs baseline: 6.9071x; 6.7168x over previous
import jax
import jax.numpy as jnp
from jax import lax
from jax.experimental import pallas as pl

_N = 4096
_K = 30
_NEG = -1e15
_R = 256
_CHUNK = 32
_NCHUNK = _N // _CHUNK


def _monotone(v):
    b = lax.bitcast_convert_type(v, jnp.int32)
    return b ^ jnp.bitwise_and(jnp.right_shift(b, 31), jnp.int32(0x7FFFFFFF))


def _topk_softmax_kernel(x_ref, o_ref):
    min32 = jnp.int32(-(2**31))
    x = x_ref[...]
    key = _monotone(x)

    def make_search(data, k):
        def run(nsteps, w0):
            def body(i, w):
                bit = jnp.left_shift(jnp.int32(1), jnp.int32(nsteps) - 1 - i)
                cand_w = jnp.bitwise_or(w, bit)
                cand_t = jnp.bitwise_xor(cand_w, min32)
                cnt = jnp.sum((data >= cand_t).astype(jnp.float32), axis=1,
                              keepdims=True)
                return jnp.where(cnt >= k, cand_w, w)

            return lax.fori_loop(0, nsteps, body, w0, unroll=8)

        return run

    cm = jnp.max(x.reshape(_R, _CHUNK, _NCHUNK), axis=1)
    keyc = _monotone(cm)
    w_t = make_search(keyc, _K)(32, jnp.zeros((_R, 1), jnp.int32))
    u_m = jnp.bitwise_xor(jnp.max(keyc, axis=1, keepdims=True), min32)
    d = jnp.bitwise_xor(w_t, u_m)
    df = lax.bitcast_convert_type(d.astype(jnp.float32), jnp.int32)
    nbits = jnp.right_shift(df, 23) - 126
    nbits = jnp.where(d < 0, jnp.int32(32), jnp.clip(nbits, 0, 32))
    shift = jnp.minimum(nbits, 31)
    pmask = jnp.where(
        nbits >= 32, jnp.int32(0),
        ~(jnp.left_shift(jnp.int32(1), shift) - 1))
    w0 = jnp.bitwise_and(w_t, pmask)
    maxnb = jnp.max(nbits)

    del maxnb
    search = make_search(key, _K)
    w = search(32, w0)
    kth = jnp.bitwise_xor(w, min32)

    keep = (key >= kth) & (x > 0.0)
    m = jnp.where(keep, x, _NEG)
    rowmax = jnp.max(m, axis=1, keepdims=True)
    e = jnp.exp(m - rowmax)
    s = jnp.sum(e, axis=1, keepdims=True)
    o_ref[...] = e / s


def kernel(new_supports):
    n = new_supports.shape[0]
    return pl.pallas_call(
        _topk_softmax_kernel,
        grid=(n // _R,),
        in_specs=[pl.BlockSpec((_R, _N), lambda i: (i, 0))],
        out_specs=pl.BlockSpec((_R, _N), lambda i: (i, 0)),
        out_shape=jax.ShapeDtypeStruct((n, _N), jnp.float32),
    )(new_supports)


# speedup vs baseline: 10.6427x; 1.5408x over previous
import jax
import jax.numpy as jnp
from jax.experimental import pallas as pl

_N = 4096
_K = 30
_NEG = -1e15
_ROWS_PER_BLOCK = 256


def _topk_softmax_kernel(x_ref, o_ref):
    x = x_ref[...]
    bi = jax.lax.bitcast_convert_type(x, jnp.int32)
    key = bi ^ jnp.bitwise_and(jnp.right_shift(bi, 31), jnp.int32(0x7FFFFFFF))
    min32 = jnp.int32(-(2**31))

    def body(i, w):
        bit = jnp.left_shift(jnp.int32(1), jnp.int32(31) - i)
        cand_w = jnp.bitwise_or(w, bit)
        cand_t = jnp.bitwise_xor(cand_w, min32)
        cnt = jnp.sum((key >= cand_t).astype(jnp.float32), axis=1, keepdims=True)
        return jnp.where(cnt >= _K, cand_w, w)

    w0 = jnp.zeros((x.shape[0], 1), jnp.int32)
    w = jax.lax.fori_loop(0, 32, body, w0, unroll=32)
    kth = jnp.bitwise_xor(w, min32)

    keep = (key >= kth) & (x > 0.0)
    m = jnp.where(keep, x, _NEG)
    rowmax = jnp.max(m, axis=1, keepdims=True)
    e = jnp.exp(m - rowmax)
    s = jnp.sum(e, axis=1, keepdims=True)
    o_ref[...] = e / s


def kernel(new_supports):
    n = new_supports.shape[0]
    r = _ROWS_PER_BLOCK
    return pl.pallas_call(
        _topk_softmax_kernel,
        grid=(n // r,),
        in_specs=[pl.BlockSpec((r, _N), lambda i: (i, 0))],
        out_specs=pl.BlockSpec((r, _N), lambda i: (i, 0)),
        out_shape=jax.ShapeDtypeStruct((n, _N), jnp.float32),
    )(new_supports)
